# speedup vs baseline: 10.4106x; 10.4106x over previous
import functools

import jax
import jax.numpy as jnp
from jax import lax
from jax.experimental import pallas as pl
from jax.experimental.pallas import tpu as pltpu
from jax.experimental.pallas import tpu_sc as plsc

C = 128
K = 16
KK = K + 1
N = 10000
NW = 32
N_PAD = 10240
PW = N_PAD // NW
CHUNK = 8
NCHUNK = PW // CHUNK
NBLK = 1024
NB = N_PAD // NBLK
L = 16


def _tc_tables_body(y_ref, w_ref, b_ref, z_ref):
    z = jax.lax.dot_general(
        y_ref[...], w_ref[0],
        (((1,), (0,)), ((), ())),
        preferred_element_type=jnp.float32,
    )
    z_ref[0] = z + b_ref[0]


def _build_tables(yT_pad, wall, ball):
    return pl.pallas_call(
        _tc_tables_body,
        grid=(NB, KK),
        in_specs=[
            pl.BlockSpec((NBLK, C), lambda nb, g: (nb, 0)),
            pl.BlockSpec((1, C, C), lambda nb, g: (g, 0, 0)),
            pl.BlockSpec((1, 1, C), lambda nb, g: (g, 0, 0)),
        ],
        out_specs=pl.BlockSpec((1, NBLK, C), lambda nb, g: (g, nb, 0)),
        out_shape=jax.ShapeDtypeStruct((KK, N_PAD, C), jnp.float32),
    )(yT_pad, wall, ball)


@functools.partial(
    pl.kernel,
    out_type=jax.ShapeDtypeStruct((N_PAD, C), jnp.float32),
    mesh=plsc.VectorSubcoreMesh(core_axis_name="c", subcore_axis_name="s"),
    scratch_types=[
        pltpu.VMEM((CHUNK, K), jnp.int32),
        pltpu.VMEM((CHUNK * K,), jnp.int32),
        pltpu.VMEM((CHUNK * K, C), jnp.float32),
        pltpu.VMEM((CHUNK, C), jnp.float32),
        pltpu.SemaphoreType.DMA,
    ],
)
def _sc_gather_sum(ztab, idxp, out, idx2_v, idxf_v, rows_v, acc_v, sem):
    wid = lax.axis_index("s") * 2 + lax.axis_index("c")
    base_pt = wid * PW
    offs = lax.iota(jnp.int32, L) * N_PAD

    def chunk_body(ci, carry):
        off = base_pt + ci * CHUNK
        pltpu.sync_copy(idxp.at[pl.ds(off, CHUNK)], idx2_v)
        for p in range(CHUNK):
            idxf_v[pl.ds(p * K, K)] = idx2_v[p, :] + offs
        gather = pltpu.async_copy(ztab.at[idxf_v], rows_v, sem)
        pltpu.sync_copy(ztab.at[pl.ds(K * N_PAD + off, CHUNK)], acc_v)
        gather.wait()
        for p in range(CHUNK):
            for cc in range(C // L):
                sl = pl.ds(cc * L, L)
                t = [rows_v[p * K + g, sl] for g in range(K)]
                t.append(acc_v[p, sl])
                while len(t) > 1:
                    nxt = [t[i] + t[i + 1] for i in range(0, len(t) - 1, 2)]
                    if len(t) % 2:
                        nxt.append(t[-1])
                    t = nxt
                acc_v[p, sl] = t[0]
        pltpu.sync_copy(acc_v, out.at[pl.ds(off, CHUNK)])
        return carry

    lax.fori_loop(0, NCHUNK, chunk_body, 0)


def kernel(x, y, y_xyz, params, idx):
    p = params
    y2 = y[0]
    wv3 = p['Wv'].reshape(C, C, K)
    a = jnp.transpose(wv3, (2, 1, 0))
    wall = jnp.concatenate([a, -a.sum(axis=0, keepdims=True)], axis=0)
    ball = jnp.zeros((KK, 1, C), jnp.float32).at[K, 0].set(p['bv'])

    yT_pad = jnp.zeros((N_PAD, C), jnp.float32).at[:N].set(y2.T)
    idxp = jnp.zeros((N_PAD, K), jnp.int32).at[:N].set(idx[0].astype(jnp.int32))

    zall = _build_tables(yT_pad, wall, ball)
    ztab = zall.reshape(KK * N_PAD, C)

    out_rows = _sc_gather_sum(ztab, idxp)
    return out_rows[:N].T[None]


# speedup vs baseline: 11.1405x; 1.0701x over previous
import functools

import jax
import jax.numpy as jnp
from jax import lax
from jax.experimental import pallas as pl
from jax.experimental.pallas import tpu as pltpu
from jax.experimental.pallas import tpu_sc as plsc

C = 128
K = 16
KK = K + 1
N = 10000
NW = 32
N_PAD = 10240
PW = N_PAD // NW
CHUNK = 8
NCHUNK = PW // CHUNK
NBLK = 1024
NB = N_PAD // NBLK
L = 16


def _tc_tables_body(y_ref, w_ref, b_ref, z_ref):
    z = jax.lax.dot_general(
        y_ref[...], w_ref[0],
        (((0,), (0,)), ((), ())),
        preferred_element_type=jnp.float32,
    )
    z_ref[0] = z + b_ref[0]


def _build_tables(y2, wall, ball):
    return pl.pallas_call(
        _tc_tables_body,
        grid=(NB, KK),
        in_specs=[
            pl.BlockSpec((C, NBLK), lambda nb, g: (0, nb)),
            pl.BlockSpec((1, C, C), lambda nb, g: (g, 0, 0)),
            pl.BlockSpec((1, 1, C), lambda nb, g: (g, 0, 0)),
        ],
        out_specs=pl.BlockSpec((1, NBLK, C), lambda nb, g: (g, nb, 0)),
        out_shape=jax.ShapeDtypeStruct((KK, N_PAD, C), jnp.float32),
    )(y2, wall, ball)


@functools.partial(
    pl.kernel,
    out_type=jax.ShapeDtypeStruct((N, C), jnp.float32),
    mesh=plsc.VectorSubcoreMesh(core_axis_name="c", subcore_axis_name="s"),
    scratch_types=[
        pltpu.VMEM((2, CHUNK, K), jnp.int32),
        pltpu.VMEM((2, CHUNK * K), jnp.int32),
        pltpu.VMEM((2, CHUNK * K, C), jnp.float32),
        pltpu.VMEM((2, CHUNK, C), jnp.float32),
        pltpu.SemaphoreType.DMA((2,)),
        pltpu.SemaphoreType.DMA((2,)),
    ],
)
def _sc_gather_sum(ztab, idx2, out, idx2_v, idxf_v, rows_v, acc_v, gsem, bsem):
    wid = lax.axis_index("s") * 2 + lax.axis_index("c")
    base_pt = wid * PW
    offs = lax.iota(jnp.int32, L) * N_PAD

    def valid(ci):
        return (ci < NCHUNK) & (base_pt + ci * CHUNK < N)

    def issue(ci, b):
        @pl.when(valid(ci))
        def _():
            off = base_pt + ci * CHUNK
            pltpu.sync_copy(idx2.at[pl.ds(off, CHUNK)], idx2_v.at[b])
            for p in range(CHUNK):
                idxf_v[b, pl.ds(p * K, K)] = idx2_v[b, p, :] + offs
            pltpu.async_copy(ztab.at[idxf_v.at[b]], rows_v.at[b], gsem.at[b])
            pltpu.async_copy(ztab.at[pl.ds(K * N_PAD + off, CHUNK)],
                             acc_v.at[b], bsem.at[b])

    issue(0, 0)
    issue(1, 1)

    def body(i, carry):
        for b in range(2):
            ci = i * 2 + b

            @pl.when(valid(ci))
            def _(b=b, ci=ci):
                off = base_pt + ci * CHUNK
                pltpu.make_async_copy(ztab.at[idxf_v.at[b]],
                                      rows_v.at[b], gsem.at[b]).wait()
                pltpu.make_async_copy(ztab.at[pl.ds(K * N_PAD + off, CHUNK)],
                                      acc_v.at[b], bsem.at[b]).wait()
                for p in range(CHUNK):
                    for cc in range(C // L):
                        sl = pl.ds(cc * L, L)
                        t = [rows_v[b, p * K + g, sl] for g in range(K)]
                        t.append(acc_v[b, p, sl])
                        while len(t) > 1:
                            nxt = [t[j] + t[j + 1]
                                   for j in range(0, len(t) - 1, 2)]
                            if len(t) % 2:
                                nxt.append(t[-1])
                            t = nxt
                        acc_v[b, p, sl] = t[0]
                pltpu.sync_copy(acc_v.at[b], out.at[pl.ds(off, CHUNK)])
                issue(ci + 2, b)
        return carry

    lax.fori_loop(0, NCHUNK // 2, body, 0)


def kernel(x, y, y_xyz, params, idx):
    p = params
    y2 = y[0]
    wv3 = p['Wv'].reshape(C, C, K)
    a = jnp.transpose(wv3, (2, 1, 0))
    wall = jnp.concatenate([a, -a.sum(axis=0, keepdims=True)], axis=0)
    ball = jnp.zeros((KK, 1, C), jnp.float32).at[K, 0].set(p['bv'])

    idx2 = idx[0].astype(jnp.int32)

    zall = _build_tables(y2, wall, ball)
    ztab = zall.reshape(KK * N_PAD, C)

    out_rows = _sc_gather_sum(ztab, idx2)
    return out_rows.T[None]
